# Initial kernel scaffold; baseline (speedup 1.0000x reference)
import jax
import jax.numpy as jnp
from jax.experimental import pallas as pl


def kernel(x, embed_table, lin_table, W1, b1, W2, b2, W3, b3):
    raise NotImplementedError("write your pallas kernel here")


# speedup vs baseline: 10.9130x; 10.9130x over previous
import functools

import jax
import jax.numpy as jnp
from jax import lax
from jax.experimental import pallas as pl
from jax.experimental.pallas import tpu as pltpu
from jax.experimental.pallas import tpu_sc as plsc

B = 16384
F = 26
E = 16
N = B * F
NW = 32
PW = N // NW
CHUNK = 128
NCH = PW // CHUNK
ROWS_PER_WORKER = NCH
D_HIDDEN = F * E


def _sc_gather(x2d, offs2d, embed_table, lin_table):
    mesh = plsc.VectorSubcoreMesh(core_axis_name="c", subcore_axis_name="s")

    @functools.partial(
        pl.kernel,
        out_type=(
            jax.ShapeDtypeStruct((N, E), jnp.float32),
            jax.ShapeDtypeStruct((N, 1), jnp.float32),
        ),
        mesh=mesh,
        scratch_types=[
            pltpu.VMEM((ROWS_PER_WORKER, CHUNK), jnp.int32),
            pltpu.VMEM((ROWS_PER_WORKER, CHUNK), jnp.int32),
            pltpu.VMEM((CHUNK, E), jnp.float32),
            pltpu.VMEM((CHUNK, 1), jnp.float32),
            pltpu.SemaphoreType.DMA,
            pltpu.SemaphoreType.DMA,
        ],
    )
    def k(x_hbm, off_hbm, emb_hbm, lin_hbm, out_e, out_l,
          idx_v, off_v, ebuf, lbuf, esem, lsem):
        wid = lax.axis_index("s") * 2 + lax.axis_index("c")
        row0 = wid * ROWS_PER_WORKER
        base = wid * PW

        pltpu.sync_copy(x_hbm.at[pl.ds(row0, ROWS_PER_WORKER)], idx_v)
        pltpu.sync_copy(off_hbm, off_v)

        def add_off(j, _):
            for kk in range(CHUNK // 16):
                sl = pl.ds(kk * 16, 16)
                idx_v[j, sl] = idx_v[j, sl] + off_v[j, sl]
            return 0

        lax.fori_loop(0, ROWS_PER_WORKER, add_off, 0)

        def chunk_body(j, _):
            pltpu.async_copy(emb_hbm.at[idx_v.at[j]], ebuf, esem).wait()
            pltpu.sync_copy(ebuf, out_e.at[pl.ds(base + j * CHUNK, CHUNK)])
            pltpu.async_copy(lin_hbm.at[idx_v.at[j]], lbuf, lsem).wait()
            pltpu.sync_copy(lbuf, out_l.at[pl.ds(base + j * CHUNK, CHUNK)])
            return 0

        lax.fori_loop(0, NCH, chunk_body, 0)

    return k(x2d, offs2d, embed_table, lin_table)


def _tc_mlp(h, linv, W1, b1, W2, b2, w3row, b3):
    BM = 1024
    grid = (B // BM,)

    def body(h_ref, l_ref, w1_ref, b1_ref, w2_ref, b2_ref, w3_ref, b3_ref,
             o_ref):
        hb = h_ref[...]
        a1 = jnp.dot(hb, w1_ref[...], preferred_element_type=jnp.float32)
        a1 = jnp.maximum(a1 + b1_ref[...], 0.0)
        a2 = jnp.dot(a1, w2_ref[...], preferred_element_type=jnp.float32)
        a2 = jnp.maximum(a2 + b2_ref[...], 0.0)
        deep = jnp.sum(a2 * w3_ref[...], axis=1) + b3_ref[0, 0]
        lin_b = jnp.sum(l_ref[...], axis=1)
        o_ref[...] = (deep + lin_b).reshape(BM // 128, 128)

    out = pl.pallas_call(
        body,
        grid=grid,
        in_specs=[
            pl.BlockSpec((BM, D_HIDDEN), lambda i: (i, 0)),
            pl.BlockSpec((BM, F), lambda i: (i, 0)),
            pl.BlockSpec((D_HIDDEN, 256), lambda i: (0, 0)),
            pl.BlockSpec((1, 256), lambda i: (0, 0)),
            pl.BlockSpec((256, 128), lambda i: (0, 0)),
            pl.BlockSpec((1, 128), lambda i: (0, 0)),
            pl.BlockSpec((1, 128), lambda i: (0, 0)),
            pl.BlockSpec((1, 1), lambda i: (0, 0)),
        ],
        out_specs=pl.BlockSpec((BM // 128, 128), lambda i: (i, 0)),
        out_shape=jax.ShapeDtypeStruct((B // 128, 128), jnp.float32),
    )(h, linv, W1, b1, W2, b2, w3row, b3)
    return out.reshape(B)


def kernel(x, embed_table, lin_table, W1, b1, W2, b2, W3, b3):
    idx = x + (jnp.arange(F, dtype=jnp.int32) * 100000)[None, :]
    h = jnp.take(embed_table, idx, axis=0).reshape(B, D_HIDDEN)
    linv = jnp.take(lin_table, idx, axis=0)[..., 0]
    return _tc_mlp(h, linv, W1, b1.reshape(1, 256), W2, b2.reshape(1, 128),
                   W3.reshape(1, 128), b3.reshape(1, 1))
